# Initial kernel scaffold; baseline (speedup 1.0000x reference)
import jax
import jax.numpy as jnp
from jax.experimental import pallas as pl


def kernel(x, edge_index, batch, W1, b1, W2, b2, W_theta, b_theta, W_h, b_h):
    raise NotImplementedError("write your pallas kernel here")


# speedup vs baseline: 13.4204x; 13.4204x over previous
import functools

import jax
import jax.numpy as jnp
from jax import lax
from jax.experimental import pallas as pl
from jax.experimental.pallas import tpu as pltpu
from jax.experimental.pallas import tpu_sc as plsc

N = 10000
E = 320000
D = 128
C = 10
G = 64

NC = 2
NS = 16
NW = NC * NS
LANES = 16

EPW = E // NW
CH = 80
NCH = EPW // CH
RPT = 624
RPT_LAST = N - RPT * (NS - 1)

DR = 640
DegRPT = DR // NS
DCH = 2000
NDCH = EPW // DCH

BT = 1000
NBT = N // BT

_HI = lax.Precision.HIGHEST
_SC_PARAMS = pltpu.CompilerParams(needs_layout_passes=False)


def _sc_degree(dst, zdeg):
    mesh = plsc.VectorSubcoreMesh(core_axis_name="c", subcore_axis_name="s")

    @functools.partial(
        pl.kernel,
        out_type=jax.ShapeDtypeStruct((NC, NS, DR, LANES), jnp.float32),
        mesh=mesh,
        scratch_types=[
            pltpu.VMEM((DCH,), jnp.int32),
            pltpu.VMEM((DR, LANES), jnp.float32),
        ],
        compiler_params=_SC_PARAMS,
    )
    def deg_kernel(dst_hbm, zdeg_hbm, out_hbm, dstv, degv):
        c = lax.axis_index("c")
        s = lax.axis_index("s")
        wid = s * NC + c
        pltpu.sync_copy(zdeg_hbm, degv)

        pos = lax.iota(jnp.int32, LANES)
        prev_i = jnp.maximum(pos - 1, 0)
        next_i = jnp.minimum(pos + 1, LANES - 1)
        base = wid * EPW

        @pl.loop(0, NDCH)
        def _chunks(chk):
            pltpu.sync_copy(dst_hbm.at[pl.ds(base + chk * DCH, DCH)], dstv)

            @pl.loop(0, DCH // LANES)
            def _vecs(j):
                d = dstv[pl.ds(j * LANES, LANES)]
                dsrt, _ = plsc.sort_key_val(d, d)
                dp = dsrt.at[prev_i].get(mode="promise_in_bounds")
                dn = dsrt.at[next_i].get(mode="promise_in_bounds")
                first = (pos == 0) | (dsrt != dp)
                last = (pos == LANES - 1) | (dsrt != dn)
                firstpos = plsc.cummax(jnp.where(first, pos, 0))
                cnt = (pos - firstpos + 1).astype(jnp.float32)
                plsc.addupdate_scatter(
                    degv,
                    [lax.shift_right_logical(dsrt, 4), lax.bitwise_and(dsrt, 15)],
                    cnt,
                    mask=last,
                )

        pltpu.sync_copy(degv, out_hbm.at[c].at[s])

    return deg_kernel(dst, zdeg)


def _sc_aggregate(table, src, dst, zrows):
    mesh = plsc.VectorSubcoreMesh(core_axis_name="c", subcore_axis_name="s")

    @functools.partial(
        pl.kernel,
        out_type=jax.ShapeDtypeStruct((NC, N, D), jnp.float32),
        mesh=mesh,
        scratch_types=[
            pltpu.VMEM((CH,), jnp.int32),
            pltpu.VMEM((CH,), jnp.int32),
            pltpu.VMEM((CH, D), jnp.float32),
            pltpu.VMEM_SHARED((N, D), jnp.float32),
        ],
    )
    def agg_kernel(table_hbm, src_hbm, dst_hbm, z_hbm, out_hbm, srcv, dstv, rows, acc):
        c = lax.axis_index("c")
        s = lax.axis_index("s")
        wid = s * NC + c

        @pl.when(s < NS - 1)
        def _():
            pltpu.sync_copy(z_hbm.at[pl.ds(0, RPT)], acc.at[pl.ds(s * RPT, RPT)])

        @pl.when(s == NS - 1)
        def _():
            pltpu.sync_copy(z_hbm, acc.at[pl.ds((NS - 1) * RPT, RPT_LAST)])

        plsc.subcore_barrier()

        base = wid * EPW

        @pl.loop(0, NCH)
        def _chunks(chk):
            off = base + chk * CH
            pltpu.sync_copy(src_hbm.at[pl.ds(off, CH)], srcv)
            pltpu.sync_copy(dst_hbm.at[pl.ds(off, CH)], dstv)
            pltpu.sync_copy(table_hbm.at[srcv], rows)
            pltpu.sync_copy(rows, acc.at[dstv], add=True)

        plsc.subcore_barrier()

        @pl.when(s < NS - 1)
        def _():
            pltpu.sync_copy(acc.at[pl.ds(s * RPT, RPT)],
                            out_hbm.at[c].at[pl.ds(s * RPT, RPT)])

        @pl.when(s == NS - 1)
        def _():
            pltpu.sync_copy(acc.at[pl.ds((NS - 1) * RPT, RPT_LAST)],
                            out_hbm.at[c].at[pl.ds((NS - 1) * RPT, RPT_LAST)])

    return agg_kernel(table, src, dst, zrows)


def _deg_factors(deg_ref):
    deg = jnp.sum(deg_ref[...], axis=1, keepdims=True) + 1.0
    dis = lax.rsqrt(deg)
    return dis, 1.0 / deg


def _tc_layer_in(x, degT, W, b2d):
    def body(x_ref, deg_ref, w_ref, b_ref, xws_ref, self_ref):
        dis, inv = _deg_factors(deg_ref)
        xw = jnp.dot(x_ref[...], w_ref[...],
                     preferred_element_type=jnp.float32, precision=_HI)
        xws_ref[...] = xw * dis
        self_ref[...] = xw * inv + b_ref[...]

    return pl.pallas_call(
        body,
        grid=(NBT,),
        in_specs=[
            pl.BlockSpec((BT, D), lambda i: (i, 0)),
            pl.BlockSpec((BT, NW), lambda i: (i, 0)),
            pl.BlockSpec((D, D), lambda i: (0, 0)),
            pl.BlockSpec((1, D), lambda i: (0, 0)),
        ],
        out_specs=[pl.BlockSpec((BT, D), lambda i: (i, 0)),
                   pl.BlockSpec((BT, D), lambda i: (i, 0))],
        out_shape=[jax.ShapeDtypeStruct((N, D), jnp.float32)] * 2,
    )(x, degT, W, b2d)


def _tc_layer_mid(p, degT, selfp, W, b2d):
    def body(p_ref, deg_ref, sp_ref, w_ref, b_ref, xws_ref, self_ref):
        dis, inv = _deg_factors(deg_ref)
        h = jnp.maximum((p_ref[0] + p_ref[1]) * dis + sp_ref[...], 0.0)
        xw = jnp.dot(h, w_ref[...],
                     preferred_element_type=jnp.float32, precision=_HI)
        xws_ref[...] = xw * dis
        self_ref[...] = xw * inv + b_ref[...]

    return pl.pallas_call(
        body,
        grid=(NBT,),
        in_specs=[
            pl.BlockSpec((NC, BT, D), lambda i: (0, i, 0)),
            pl.BlockSpec((BT, NW), lambda i: (i, 0)),
            pl.BlockSpec((BT, D), lambda i: (i, 0)),
            pl.BlockSpec((D, D), lambda i: (0, 0)),
            pl.BlockSpec((1, D), lambda i: (0, 0)),
        ],
        out_specs=[pl.BlockSpec((BT, D), lambda i: (i, 0)),
                   pl.BlockSpec((BT, D), lambda i: (i, 0))],
        out_shape=[jax.ShapeDtypeStruct((N, D), jnp.float32)] * 2,
    )(p, degT, selfp, W, b2d)


def _tc_final(q, degT, selfp, Wt, bt, Wh, bh, batch3):
    def body(q_ref, deg_ref, sp_ref, wt_ref, bt_ref, wh_ref, bh_ref, b_ref,
             h_ref, th_ref, hc_ref, lsm_ref, pooled):
        i = pl.program_id(0)
        dis, _ = _deg_factors(deg_ref)
        h2 = (q_ref[0] + q_ref[1]) * dis + sp_ref[...]
        h_ref[...] = h2
        theta = jnp.dot(h2, wt_ref[...],
                        preferred_element_type=jnp.float32, precision=_HI) + bt_ref[...]
        hc = jnp.dot(h2, wh_ref[...],
                     preferred_element_type=jnp.float32, precision=_HI) + bh_ref[...]
        th_ref[...] = theta[:, :C]
        hc_ref[...] = hc[:, :C]
        prod = theta * hc
        bb = b_ref[0]
        gi = lax.broadcasted_iota(jnp.int32, (G, BT), 0)
        oh = (gi == bb).astype(jnp.float32)
        part = jnp.dot(oh, prod, preferred_element_type=jnp.float32, precision=_HI)

        @pl.when(i == 0)
        def _():
            pooled[...] = jnp.zeros((G, D), jnp.float32)

        pooled[...] += part
        z = pooled[...]
        col = lax.broadcasted_iota(jnp.int32, (G, D), 1)
        zm = jnp.where(col < C, z, -jnp.inf)
        m = jnp.max(zm, axis=1, keepdims=True)
        ssum = jnp.sum(jnp.exp(zm - m), axis=1, keepdims=True)
        lsm_ref[...] = (z - m - jnp.log(ssum))[:, :C]

    return pl.pallas_call(
        body,
        grid=(NBT,),
        in_specs=[
            pl.BlockSpec((NC, BT, D), lambda i: (0, i, 0)),
            pl.BlockSpec((BT, NW), lambda i: (i, 0)),
            pl.BlockSpec((BT, D), lambda i: (i, 0)),
            pl.BlockSpec((D, D), lambda i: (0, 0)),
            pl.BlockSpec((1, D), lambda i: (0, 0)),
            pl.BlockSpec((D, D), lambda i: (0, 0)),
            pl.BlockSpec((1, D), lambda i: (0, 0)),
            pl.BlockSpec((1, 1, BT), lambda i: (i, 0, 0)),
        ],
        out_specs=[
            pl.BlockSpec((BT, D), lambda i: (i, 0)),
            pl.BlockSpec((BT, C), lambda i: (i, 0)),
            pl.BlockSpec((BT, C), lambda i: (i, 0)),
            pl.BlockSpec((G, C), lambda i: (0, 0)),
        ],
        out_shape=[
            jax.ShapeDtypeStruct((N, D), jnp.float32),
            jax.ShapeDtypeStruct((N, C), jnp.float32),
            jax.ShapeDtypeStruct((N, C), jnp.float32),
            jax.ShapeDtypeStruct((G, C), jnp.float32),
        ],
        scratch_shapes=[pltpu.VMEM((G, D), jnp.float32)],
    )(q, degT, selfp, Wt, bt, Wh, bh, batch3)


def kernel(x, edge_index, batch, W1, b1, W2, b2, W_theta, b_theta, W_h, b_h):
    f32 = jnp.float32
    src = edge_index[0].astype(jnp.int32)
    dst = edge_index[1].astype(jnp.int32)
    zdeg = jnp.zeros((DR, LANES), f32)
    zrows = jnp.zeros((RPT_LAST, D), f32)

    deg2 = _sc_degree(dst, zdeg)
    degT = deg2.reshape(NW, DR * LANES)[:, :N].T

    xws1, self1 = _tc_layer_in(x, degT, W1, b1.reshape(1, D))
    p = _sc_aggregate(xws1, src, dst, zrows)
    xws2, self2 = _tc_layer_mid(p, degT, self1, W2, b2.reshape(1, D))
    q = _sc_aggregate(xws2, src, dst, zrows)

    Wt = jnp.zeros((D, D), f32).at[:, :C].set(W_theta)
    Wh = jnp.zeros((D, D), f32).at[:, :C].set(W_h)
    bt = jnp.zeros((1, D), f32).at[0, :C].set(b_theta)
    bh = jnp.zeros((1, D), f32).at[0, :C].set(b_h)
    batch3 = batch.astype(jnp.int32).reshape(NBT, 1, BT)

    h, theta, hc, lsm = _tc_final(q, degT, self2, Wt, bt, Wh, bh, batch3)
    return lsm, h, theta, hc


# speedup vs baseline: 23.2801x; 1.7347x over previous
import functools

import jax
import jax.numpy as jnp
from jax import lax
from jax.experimental import pallas as pl
from jax.experimental.pallas import tpu as pltpu
from jax.experimental.pallas import tpu_sc as plsc

N = 10000
E = 320000
D = 128
C = 10
G = 64

NC = 2
NS = 16
NW = NC * NS
LANES = 16

EPW = E // NW
EPAD = 240
EPW2 = EPW + EPAD
CH = 128
NCH = EPW2 // CH
ACC_ROWS = 10240
RPT = 624
RPT_LAST = N - RPT * (NS - 1)

DR = 640
DegRPT = DR // NS
DCH = 2000
NDCH = EPW // DCH

BT = 1000
NBT = N // BT

_HI = lax.Precision.HIGHEST
_SC_PARAMS = pltpu.CompilerParams(needs_layout_passes=False)


def _sc_degree(dst, zdeg):
    mesh = plsc.VectorSubcoreMesh(core_axis_name="c", subcore_axis_name="s")

    @functools.partial(
        pl.kernel,
        out_type=jax.ShapeDtypeStruct((NC, NS, DR, LANES), jnp.float32),
        mesh=mesh,
        scratch_types=[
            pltpu.VMEM((DCH,), jnp.int32),
            pltpu.VMEM((DR, LANES), jnp.float32),
        ],
        compiler_params=_SC_PARAMS,
    )
    def deg_kernel(dst_hbm, zdeg_hbm, out_hbm, dstv, degv):
        c = lax.axis_index("c")
        s = lax.axis_index("s")
        wid = s * NC + c
        pltpu.sync_copy(zdeg_hbm, degv)

        pos = lax.iota(jnp.int32, LANES)
        prev_i = jnp.maximum(pos - 1, 0)
        next_i = jnp.minimum(pos + 1, LANES - 1)
        base = wid * EPW

        @pl.loop(0, NDCH)
        def _chunks(chk):
            pltpu.sync_copy(dst_hbm.at[pl.ds(base + chk * DCH, DCH)], dstv)

            @pl.loop(0, DCH // LANES)
            def _vecs(j):
                d = dstv[pl.ds(j * LANES, LANES)]
                dsrt, _ = plsc.sort_key_val(d, d)
                dp = dsrt.at[prev_i].get(mode="promise_in_bounds")
                dn = dsrt.at[next_i].get(mode="promise_in_bounds")
                first = (pos == 0) | (dsrt != dp)
                last = (pos == LANES - 1) | (dsrt != dn)
                firstpos = plsc.cummax(jnp.where(first, pos, 0))
                cnt = (pos - firstpos + 1).astype(jnp.float32)
                plsc.addupdate_scatter(
                    degv,
                    [lax.shift_right_logical(dsrt, 4), lax.bitwise_and(dsrt, 15)],
                    cnt,
                    mask=last,
                )

        pltpu.sync_copy(degv, out_hbm.at[c].at[s])

    return deg_kernel(dst, zdeg)


def _sc_aggregate(table, src, dst, zrows):
    mesh = plsc.VectorSubcoreMesh(core_axis_name="c", subcore_axis_name="s")

    @functools.partial(
        pl.kernel,
        out_type=jax.ShapeDtypeStruct((NC, N, D), jnp.float32),
        mesh=mesh,
        scratch_types=[
            pltpu.VMEM((2 * CH,), jnp.int32),
            pltpu.VMEM((CH,), jnp.int32),
            pltpu.VMEM((CH,), jnp.int32),
            pltpu.VMEM((CH, D), jnp.float32),
            pltpu.VMEM((CH, D), jnp.float32),
            pltpu.SemaphoreType.DMA,
            pltpu.SemaphoreType.DMA,
            pltpu.SemaphoreType.DMA,
            pltpu.SemaphoreType.DMA,
            pltpu.VMEM_SHARED((ACC_ROWS, D), jnp.float32),
        ],
    )
    def agg_kernel(table_hbm, src_hbm, dst_hbm, z_hbm, out_hbm,
                   srcv, didx0, didx1, rows0, rows1,
                   gsem0, gsem1, ssem0, ssem1, acc):
        c = lax.axis_index("c")
        s = lax.axis_index("s")
        wid = s * NC + c

        @pl.when(s < NS - 1)
        def _():
            pltpu.sync_copy(z_hbm.at[pl.ds(0, RPT)], acc.at[pl.ds(s * RPT, RPT)])

        @pl.when(s == NS - 1)
        def _():
            pltpu.sync_copy(z_hbm, acc.at[pl.ds((NS - 1) * RPT, RPT_LAST)])

        plsc.subcore_barrier()

        base = wid * EPW2

        @pl.loop(0, NCH, step=2)
        def _chunks(k):
            off = base + k * CH
            pltpu.sync_copy(src_hbm.at[pl.ds(off, 2 * CH)], srcv)

            @pl.when(k > 0)
            def _():
                pltpu.make_async_copy(rows0, acc.at[didx0], ssem0).wait()

            pltpu.sync_copy(dst_hbm.at[pl.ds(off, CH)], didx0)
            g0 = pltpu.async_copy(table_hbm.at[srcv.at[pl.ds(0, CH)]],
                                  rows0, gsem0)

            @pl.when(k > 0)
            def _():
                pltpu.make_async_copy(rows1, acc.at[didx1], ssem1).wait()

            pltpu.sync_copy(dst_hbm.at[pl.ds(off + CH, CH)], didx1)
            g1 = pltpu.async_copy(table_hbm.at[srcv.at[pl.ds(CH, CH)]],
                                  rows1, gsem1)

            g0.wait()
            pltpu.async_copy(rows0, acc.at[didx0], ssem0, add=True)
            g1.wait()
            pltpu.async_copy(rows1, acc.at[didx1], ssem1, add=True)

        pltpu.make_async_copy(rows0, acc.at[didx0], ssem0).wait()
        pltpu.make_async_copy(rows1, acc.at[didx1], ssem1).wait()
        plsc.subcore_barrier()

        @pl.when(s < NS - 1)
        def _():
            pltpu.sync_copy(acc.at[pl.ds(s * RPT, RPT)],
                            out_hbm.at[c].at[pl.ds(s * RPT, RPT)])

        @pl.when(s == NS - 1)
        def _():
            pltpu.sync_copy(acc.at[pl.ds((NS - 1) * RPT, RPT_LAST)],
                            out_hbm.at[c].at[pl.ds((NS - 1) * RPT, RPT_LAST)])

    return agg_kernel(table, src, dst, zrows)


def _deg_factors(deg_ref):
    deg = jnp.sum(deg_ref[...], axis=1, keepdims=True) + 1.0
    dis = lax.rsqrt(deg)
    return dis, 1.0 / deg


def _tc_layer_in(x, degT, W, b2d):
    def body(x_ref, deg_ref, w_ref, b_ref, xws_ref, self_ref):
        dis, inv = _deg_factors(deg_ref)
        xw = jnp.dot(x_ref[...], w_ref[...],
                     preferred_element_type=jnp.float32, precision=_HI)
        xws_ref[...] = xw * dis
        self_ref[...] = xw * inv + b_ref[...]

    return pl.pallas_call(
        body,
        grid=(NBT,),
        in_specs=[
            pl.BlockSpec((BT, D), lambda i: (i, 0)),
            pl.BlockSpec((BT, NW), lambda i: (i, 0)),
            pl.BlockSpec((D, D), lambda i: (0, 0)),
            pl.BlockSpec((1, D), lambda i: (0, 0)),
        ],
        out_specs=[pl.BlockSpec((BT, D), lambda i: (i, 0)),
                   pl.BlockSpec((BT, D), lambda i: (i, 0))],
        out_shape=[jax.ShapeDtypeStruct((N, D), jnp.float32)] * 2,
    )(x, degT, W, b2d)


def _tc_layer_mid(p, degT, selfp, W, b2d):
    def body(p_ref, deg_ref, sp_ref, w_ref, b_ref, xws_ref, self_ref):
        dis, inv = _deg_factors(deg_ref)
        h = jnp.maximum((p_ref[0] + p_ref[1]) * dis + sp_ref[...], 0.0)
        xw = jnp.dot(h, w_ref[...],
                     preferred_element_type=jnp.float32, precision=_HI)
        xws_ref[...] = xw * dis
        self_ref[...] = xw * inv + b_ref[...]

    return pl.pallas_call(
        body,
        grid=(NBT,),
        in_specs=[
            pl.BlockSpec((NC, BT, D), lambda i: (0, i, 0)),
            pl.BlockSpec((BT, NW), lambda i: (i, 0)),
            pl.BlockSpec((BT, D), lambda i: (i, 0)),
            pl.BlockSpec((D, D), lambda i: (0, 0)),
            pl.BlockSpec((1, D), lambda i: (0, 0)),
        ],
        out_specs=[pl.BlockSpec((BT, D), lambda i: (i, 0)),
                   pl.BlockSpec((BT, D), lambda i: (i, 0))],
        out_shape=[jax.ShapeDtypeStruct((N, D), jnp.float32)] * 2,
    )(p, degT, selfp, W, b2d)


def _tc_final(q, degT, selfp, Wt, bt, Wh, bh, batch3):
    def body(q_ref, deg_ref, sp_ref, wt_ref, bt_ref, wh_ref, bh_ref, b_ref,
             h_ref, th_ref, hc_ref, lsm_ref, pooled):
        i = pl.program_id(0)
        dis, _ = _deg_factors(deg_ref)
        h2 = (q_ref[0] + q_ref[1]) * dis + sp_ref[...]
        h_ref[...] = h2
        theta = jnp.dot(h2, wt_ref[...],
                        preferred_element_type=jnp.float32, precision=_HI) + bt_ref[...]
        hc = jnp.dot(h2, wh_ref[...],
                     preferred_element_type=jnp.float32, precision=_HI) + bh_ref[...]
        th_ref[...] = theta[:, :C]
        hc_ref[...] = hc[:, :C]
        prod = theta * hc
        bb = b_ref[0]
        gi = lax.broadcasted_iota(jnp.int32, (G, BT), 0)
        oh = (gi == bb).astype(jnp.float32)
        part = jnp.dot(oh, prod, preferred_element_type=jnp.float32, precision=_HI)

        @pl.when(i == 0)
        def _():
            pooled[...] = jnp.zeros((G, D), jnp.float32)

        pooled[...] += part
        z = pooled[...]
        col = lax.broadcasted_iota(jnp.int32, (G, D), 1)
        zm = jnp.where(col < C, z, -jnp.inf)
        m = jnp.max(zm, axis=1, keepdims=True)
        ssum = jnp.sum(jnp.exp(zm - m), axis=1, keepdims=True)
        lsm_ref[...] = (z - m - jnp.log(ssum))[:, :C]

    return pl.pallas_call(
        body,
        grid=(NBT,),
        in_specs=[
            pl.BlockSpec((NC, BT, D), lambda i: (0, i, 0)),
            pl.BlockSpec((BT, NW), lambda i: (i, 0)),
            pl.BlockSpec((BT, D), lambda i: (i, 0)),
            pl.BlockSpec((D, D), lambda i: (0, 0)),
            pl.BlockSpec((1, D), lambda i: (0, 0)),
            pl.BlockSpec((D, D), lambda i: (0, 0)),
            pl.BlockSpec((1, D), lambda i: (0, 0)),
            pl.BlockSpec((1, 1, BT), lambda i: (i, 0, 0)),
        ],
        out_specs=[
            pl.BlockSpec((BT, D), lambda i: (i, 0)),
            pl.BlockSpec((BT, C), lambda i: (i, 0)),
            pl.BlockSpec((BT, C), lambda i: (i, 0)),
            pl.BlockSpec((G, C), lambda i: (0, 0)),
        ],
        out_shape=[
            jax.ShapeDtypeStruct((N, D), jnp.float32),
            jax.ShapeDtypeStruct((N, C), jnp.float32),
            jax.ShapeDtypeStruct((N, C), jnp.float32),
            jax.ShapeDtypeStruct((G, C), jnp.float32),
        ],
        scratch_shapes=[pltpu.VMEM((G, D), jnp.float32)],
    )(q, degT, selfp, Wt, bt, Wh, bh, batch3)


def kernel(x, edge_index, batch, W1, b1, W2, b2, W_theta, b_theta, W_h, b_h):
    f32 = jnp.float32
    src = edge_index[0].astype(jnp.int32)
    dst = edge_index[1].astype(jnp.int32)
    zdeg = jnp.zeros((DR, LANES), f32)
    zrows = jnp.zeros((RPT_LAST, D), f32)

    deg2 = _sc_degree(dst, zdeg)
    degT = deg2.reshape(NW, DR * LANES)[:, :N].T

    pad_src = jnp.broadcast_to(jnp.arange(EPAD, dtype=jnp.int32)[None, :],
                               (NW, EPAD))
    pad_dst = jnp.broadcast_to((N + jnp.arange(EPAD, dtype=jnp.int32))[None, :],
                               (NW, EPAD))
    src_p = jnp.concatenate([src.reshape(NW, EPW), pad_src], axis=1).reshape(-1)
    dst_p = jnp.concatenate([dst.reshape(NW, EPW), pad_dst], axis=1).reshape(-1)

    xws1, self1 = _tc_layer_in(x, degT, W1, b1.reshape(1, D))
    p = _sc_aggregate(xws1, src_p, dst_p, zrows)
    xws2, self2 = _tc_layer_mid(p, degT, self1, W2, b2.reshape(1, D))
    q = _sc_aggregate(xws2, src_p, dst_p, zrows)

    Wt = jnp.zeros((D, D), f32).at[:, :C].set(W_theta)
    Wh = jnp.zeros((D, D), f32).at[:, :C].set(W_h)
    bt = jnp.zeros((1, D), f32).at[0, :C].set(b_theta)
    bh = jnp.zeros((1, D), f32).at[0, :C].set(b_h)
    batch3 = batch.astype(jnp.int32).reshape(NBT, 1, BT)

    h, theta, hc, lsm = _tc_final(q, degT, self2, Wt, bt, Wh, bh, batch3)
    return lsm, h, theta, hc


# speedup vs baseline: 25.0414x; 1.0757x over previous
import functools

import jax
import jax.numpy as jnp
from jax import lax
from jax.experimental import pallas as pl
from jax.experimental.pallas import tpu as pltpu
from jax.experimental.pallas import tpu_sc as plsc

N = 10000
E = 320000
D = 128
C = 10
G = 64

NC = 2
NS = 16
NW = NC * NS
LANES = 16

EPW = E // NW
EPAD = 240
EPW2 = EPW + EPAD
CH = 128
NCH = EPW2 // CH
ACC_ROWS = 10240
RPT = 624
RPT_LAST = N - RPT * (NS - 1)

DR = 640
DegRPT = DR // NS
DCH = 2000
NDCH = EPW // DCH

BT = 1000
NBT = N // BT

_HI = lax.Precision.HIGHEST
_SC_PARAMS = pltpu.CompilerParams(needs_layout_passes=False)


def _sc_degree(dst, zdeg):
    mesh = plsc.VectorSubcoreMesh(core_axis_name="c", subcore_axis_name="s")

    @functools.partial(
        pl.kernel,
        out_type=jax.ShapeDtypeStruct((NC, NS, DR, LANES), jnp.float32),
        mesh=mesh,
        scratch_types=[
            pltpu.VMEM((DCH,), jnp.int32),
            pltpu.VMEM((DR, LANES), jnp.float32),
        ],
        compiler_params=_SC_PARAMS,
    )
    def deg_kernel(dst_hbm, zdeg_hbm, out_hbm, dstv, degv):
        c = lax.axis_index("c")
        s = lax.axis_index("s")
        wid = s * NC + c
        pltpu.sync_copy(zdeg_hbm, degv)

        pos = lax.iota(jnp.int32, LANES)
        prev_i = jnp.maximum(pos - 1, 0)
        next_i = jnp.minimum(pos + 1, LANES - 1)
        base = wid * EPW

        @pl.loop(0, NDCH)
        def _chunks(chk):
            pltpu.sync_copy(dst_hbm.at[pl.ds(base + chk * DCH, DCH)], dstv)

            @pl.loop(0, DCH // LANES)
            def _vecs(j):
                d = dstv[pl.ds(j * LANES, LANES)]
                dsrt, _ = plsc.sort_key_val(d, d)
                dp = dsrt.at[prev_i].get(mode="promise_in_bounds")
                dn = dsrt.at[next_i].get(mode="promise_in_bounds")
                first = (pos == 0) | (dsrt != dp)
                last = (pos == LANES - 1) | (dsrt != dn)
                firstpos = plsc.cummax(jnp.where(first, pos, 0))
                cnt = (pos - firstpos + 1).astype(jnp.float32)
                plsc.addupdate_scatter(
                    degv,
                    [lax.shift_right_logical(dsrt, 4), lax.bitwise_and(dsrt, 15)],
                    cnt,
                    mask=last,
                )

        pltpu.sync_copy(degv, out_hbm.at[c].at[s])

    return deg_kernel(dst, zdeg)


def _sc_aggregate(table, src, dst, zrows):
    mesh = plsc.VectorSubcoreMesh(core_axis_name="c", subcore_axis_name="s")

    @functools.partial(
        pl.kernel,
        out_type=jax.ShapeDtypeStruct((NC, N, D), jnp.float32),
        mesh=mesh,
        scratch_types=(
            [pltpu.VMEM((CH,), jnp.int32)] * 4
            + [pltpu.VMEM((CH,), jnp.int32)] * 4
            + [pltpu.VMEM((CH, D), jnp.float32)] * 2
            + [pltpu.SemaphoreType.DMA] * 12
            + [pltpu.VMEM_SHARED((ACC_ROWS, D), jnp.float32)]
        ),
    )
    def agg_kernel(table_hbm, src_hbm, dst_hbm, z_hbm, out_hbm,
                   sv0, sv1, sv2, sv3, dv0, dv1, dv2, dv3,
                   rows0, rows1,
                   is0, is1, is2, is3, js0, js1, js2, js3,
                   gsem0, gsem1, ssem0, ssem1, acc):
        c = lax.axis_index("c")
        s = lax.axis_index("s")
        wid = s * NC + c
        svs = (sv0, sv1, sv2, sv3)
        dvs = (dv0, dv1, dv2, dv3)
        isems = (is0, is1, is2, is3)
        jsems = (js0, js1, js2, js3)
        rows = (rows0, rows1)
        gsems = (gsem0, gsem1)
        ssems = (ssem0, ssem1)

        @pl.when(s < NS - 1)
        def _():
            pltpu.sync_copy(z_hbm.at[pl.ds(0, RPT)], acc.at[pl.ds(s * RPT, RPT)])

        @pl.when(s == NS - 1)
        def _():
            pltpu.sync_copy(z_hbm, acc.at[pl.ds((NS - 1) * RPT, RPT_LAST)])

        plsc.subcore_barrier()

        base = wid * EPW2

        def issue_idx(chunk, slot):
            off = base + chunk * CH
            pltpu.async_copy(src_hbm.at[pl.ds(off, CH)], svs[slot], isems[slot])
            pltpu.async_copy(dst_hbm.at[pl.ds(off, CH)], dvs[slot], jsems[slot])

        issue_idx(0, 0)
        issue_idx(1, 1)

        @pl.loop(0, NCH, step=4)
        def _grp(k):
            for u in range(4):
                b2, b4 = u % 2, u
                pltpu.make_async_copy(src_hbm.at[pl.ds(0, CH)],
                                      svs[b4], isems[b4]).wait()
                pltpu.make_async_copy(dst_hbm.at[pl.ds(0, CH)],
                                      dvs[b4], jsems[b4]).wait()

                def wait_scatter(bb=b2):
                    pltpu.make_async_copy(rows[bb], acc.at[dvs[bb]],
                                          ssems[bb]).wait()
                if u < 2:
                    pl.when(k > 0)(wait_scatter)
                else:
                    wait_scatter()

                pltpu.async_copy(table_hbm.at[svs[b4]], rows[b2], gsems[b2])

                def prefetch(ch=k + u + 2, slot=(u + 2) % 4):
                    issue_idx(ch, slot)
                if u < 2:
                    prefetch()
                else:
                    pl.when(k < NCH - 4)(prefetch)

                pltpu.make_async_copy(table_hbm.at[svs[b4]],
                                      rows[b2], gsems[b2]).wait()
                pltpu.async_copy(rows[b2], acc.at[dvs[b4]], ssems[b2], add=True)

        pltpu.make_async_copy(rows0, acc.at[dv2], ssem0).wait()
        pltpu.make_async_copy(rows1, acc.at[dv3], ssem1).wait()
        plsc.subcore_barrier()

        @pl.when(s < NS - 1)
        def _():
            pltpu.sync_copy(acc.at[pl.ds(s * RPT, RPT)],
                            out_hbm.at[c].at[pl.ds(s * RPT, RPT)])

        @pl.when(s == NS - 1)
        def _():
            pltpu.sync_copy(acc.at[pl.ds((NS - 1) * RPT, RPT_LAST)],
                            out_hbm.at[c].at[pl.ds((NS - 1) * RPT, RPT_LAST)])

    return agg_kernel(table, src, dst, zrows)


def _deg_factors(deg_ref):
    deg = jnp.sum(deg_ref[...], axis=1, keepdims=True) + 1.0
    dis = lax.rsqrt(deg)
    return dis, 1.0 / deg


def _tc_layer_in(x, degT, W, b2d):
    def body(x_ref, deg_ref, w_ref, b_ref, xws_ref, self_ref):
        dis, inv = _deg_factors(deg_ref)
        xw = jnp.dot(x_ref[...], w_ref[...],
                     preferred_element_type=jnp.float32, precision=_HI)
        xws_ref[...] = xw * dis
        self_ref[...] = xw * inv + b_ref[...]

    return pl.pallas_call(
        body,
        grid=(NBT,),
        in_specs=[
            pl.BlockSpec((BT, D), lambda i: (i, 0)),
            pl.BlockSpec((BT, NW), lambda i: (i, 0)),
            pl.BlockSpec((D, D), lambda i: (0, 0)),
            pl.BlockSpec((1, D), lambda i: (0, 0)),
        ],
        out_specs=[pl.BlockSpec((BT, D), lambda i: (i, 0)),
                   pl.BlockSpec((BT, D), lambda i: (i, 0))],
        out_shape=[jax.ShapeDtypeStruct((N, D), jnp.float32)] * 2,
    )(x, degT, W, b2d)


def _tc_layer_mid(p, degT, selfp, W, b2d):
    def body(p_ref, deg_ref, sp_ref, w_ref, b_ref, xws_ref, self_ref):
        dis, inv = _deg_factors(deg_ref)
        h = jnp.maximum((p_ref[0] + p_ref[1]) * dis + sp_ref[...], 0.0)
        xw = jnp.dot(h, w_ref[...],
                     preferred_element_type=jnp.float32, precision=_HI)
        xws_ref[...] = xw * dis
        self_ref[...] = xw * inv + b_ref[...]

    return pl.pallas_call(
        body,
        grid=(NBT,),
        in_specs=[
            pl.BlockSpec((NC, BT, D), lambda i: (0, i, 0)),
            pl.BlockSpec((BT, NW), lambda i: (i, 0)),
            pl.BlockSpec((BT, D), lambda i: (i, 0)),
            pl.BlockSpec((D, D), lambda i: (0, 0)),
            pl.BlockSpec((1, D), lambda i: (0, 0)),
        ],
        out_specs=[pl.BlockSpec((BT, D), lambda i: (i, 0)),
                   pl.BlockSpec((BT, D), lambda i: (i, 0))],
        out_shape=[jax.ShapeDtypeStruct((N, D), jnp.float32)] * 2,
    )(p, degT, selfp, W, b2d)


def _tc_final(q, degT, selfp, Wt, bt, Wh, bh, batch3):
    def body(q_ref, deg_ref, sp_ref, wt_ref, bt_ref, wh_ref, bh_ref, b_ref,
             h_ref, th_ref, hc_ref, lsm_ref, pooled):
        i = pl.program_id(0)
        dis, _ = _deg_factors(deg_ref)
        h2 = (q_ref[0] + q_ref[1]) * dis + sp_ref[...]
        h_ref[...] = h2
        theta = jnp.dot(h2, wt_ref[...],
                        preferred_element_type=jnp.float32, precision=_HI) + bt_ref[...]
        hc = jnp.dot(h2, wh_ref[...],
                     preferred_element_type=jnp.float32, precision=_HI) + bh_ref[...]
        th_ref[...] = theta[:, :C]
        hc_ref[...] = hc[:, :C]
        prod = theta * hc
        bb = b_ref[0]
        gi = lax.broadcasted_iota(jnp.int32, (G, BT), 0)
        oh = (gi == bb).astype(jnp.float32)
        part = jnp.dot(oh, prod, preferred_element_type=jnp.float32, precision=_HI)

        @pl.when(i == 0)
        def _():
            pooled[...] = jnp.zeros((G, D), jnp.float32)

        pooled[...] += part
        z = pooled[...]
        col = lax.broadcasted_iota(jnp.int32, (G, D), 1)
        zm = jnp.where(col < C, z, -jnp.inf)
        m = jnp.max(zm, axis=1, keepdims=True)
        ssum = jnp.sum(jnp.exp(zm - m), axis=1, keepdims=True)
        lsm_ref[...] = (z - m - jnp.log(ssum))[:, :C]

    return pl.pallas_call(
        body,
        grid=(NBT,),
        in_specs=[
            pl.BlockSpec((NC, BT, D), lambda i: (0, i, 0)),
            pl.BlockSpec((BT, NW), lambda i: (i, 0)),
            pl.BlockSpec((BT, D), lambda i: (i, 0)),
            pl.BlockSpec((D, D), lambda i: (0, 0)),
            pl.BlockSpec((1, D), lambda i: (0, 0)),
            pl.BlockSpec((D, D), lambda i: (0, 0)),
            pl.BlockSpec((1, D), lambda i: (0, 0)),
            pl.BlockSpec((1, 1, BT), lambda i: (i, 0, 0)),
        ],
        out_specs=[
            pl.BlockSpec((BT, D), lambda i: (i, 0)),
            pl.BlockSpec((BT, C), lambda i: (i, 0)),
            pl.BlockSpec((BT, C), lambda i: (i, 0)),
            pl.BlockSpec((G, C), lambda i: (0, 0)),
        ],
        out_shape=[
            jax.ShapeDtypeStruct((N, D), jnp.float32),
            jax.ShapeDtypeStruct((N, C), jnp.float32),
            jax.ShapeDtypeStruct((N, C), jnp.float32),
            jax.ShapeDtypeStruct((G, C), jnp.float32),
        ],
        scratch_shapes=[pltpu.VMEM((G, D), jnp.float32)],
    )(q, degT, selfp, Wt, bt, Wh, bh, batch3)


def kernel(x, edge_index, batch, W1, b1, W2, b2, W_theta, b_theta, W_h, b_h):
    f32 = jnp.float32
    src = edge_index[0].astype(jnp.int32)
    dst = edge_index[1].astype(jnp.int32)
    zdeg = jnp.zeros((DR, LANES), f32)
    zrows = jnp.zeros((RPT_LAST, D), f32)

    deg2 = _sc_degree(dst, zdeg)
    degT = deg2.reshape(NW, DR * LANES)[:, :N].T

    pad_src = jnp.broadcast_to(jnp.arange(EPAD, dtype=jnp.int32)[None, :],
                               (NW, EPAD))
    pad_dst = jnp.broadcast_to((N + jnp.arange(EPAD, dtype=jnp.int32))[None, :],
                               (NW, EPAD))
    src_p = jnp.concatenate([src.reshape(NW, EPW), pad_src], axis=1).reshape(-1)
    dst_p = jnp.concatenate([dst.reshape(NW, EPW), pad_dst], axis=1).reshape(-1)

    xws1, self1 = _tc_layer_in(x, degT, W1, b1.reshape(1, D))
    p = _sc_aggregate(xws1, src_p, dst_p, zrows)
    xws2, self2 = _tc_layer_mid(p, degT, self1, W2, b2.reshape(1, D))
    q = _sc_aggregate(xws2, src_p, dst_p, zrows)

    Wt = jnp.zeros((D, D), f32).at[:, :C].set(W_theta)
    Wh = jnp.zeros((D, D), f32).at[:, :C].set(W_h)
    bt = jnp.zeros((1, D), f32).at[0, :C].set(b_theta)
    bh = jnp.zeros((1, D), f32).at[0, :C].set(b_h)
    batch3 = batch.astype(jnp.int32).reshape(NBT, 1, BT)

    h, theta, hc, lsm = _tc_final(q, degT, self2, Wt, bt, Wh, bh, batch3)
    return lsm, h, theta, hc


# speedup vs baseline: 25.0762x; 1.0014x over previous
import functools

import jax
import jax.numpy as jnp
from jax import lax
from jax.experimental import pallas as pl
from jax.experimental.pallas import tpu as pltpu
from jax.experimental.pallas import tpu_sc as plsc

N = 10000
E = 320000
D = 128
C = 10
G = 64

NC = 2
NS = 16
NW = NC * NS
LANES = 16

EPW = E // NW
EPAD = 240
EPW2 = EPW + EPAD
CH = 128
NCH = EPW2 // CH
ACC_ROWS = 10240
RPT = 624
RPT_LAST = N - RPT * (NS - 1)

DR = 640
DegRPT = DR // NS
DCH = 2000
NDCH = EPW // DCH

BT = 1000
NBT = N // BT

_HI = lax.Precision.HIGHEST
_SC_PARAMS = pltpu.CompilerParams(needs_layout_passes=False)


def _sc_degree(dst, zdeg):
    mesh = plsc.VectorSubcoreMesh(core_axis_name="c", subcore_axis_name="s")

    @functools.partial(
        pl.kernel,
        out_type=jax.ShapeDtypeStruct((NC, NS, DR, LANES), jnp.float32),
        mesh=mesh,
        scratch_types=[
            pltpu.VMEM((DCH,), jnp.int32),
            pltpu.VMEM((DR, LANES), jnp.float32),
        ],
        compiler_params=_SC_PARAMS,
    )
    def deg_kernel(dst_hbm, zdeg_hbm, out_hbm, dstv, degv):
        c = lax.axis_index("c")
        s = lax.axis_index("s")
        wid = s * NC + c
        pltpu.sync_copy(zdeg_hbm, degv)

        pos = lax.iota(jnp.int32, LANES)
        prev_i = jnp.maximum(pos - 1, 0)
        next_i = jnp.minimum(pos + 1, LANES - 1)
        base = wid * EPW

        @pl.loop(0, NDCH)
        def _chunks(chk):
            pltpu.sync_copy(dst_hbm.at[pl.ds(base + chk * DCH, DCH)], dstv)

            @pl.loop(0, DCH // LANES)
            def _vecs(j):
                d = dstv[pl.ds(j * LANES, LANES)]
                dsrt, _ = plsc.sort_key_val(d, d)
                dp = dsrt.at[prev_i].get(mode="promise_in_bounds")
                dn = dsrt.at[next_i].get(mode="promise_in_bounds")
                first = (pos == 0) | (dsrt != dp)
                last = (pos == LANES - 1) | (dsrt != dn)
                firstpos = plsc.cummax(jnp.where(first, pos, 0))
                cnt = (pos - firstpos + 1).astype(jnp.float32)
                plsc.addupdate_scatter(
                    degv,
                    [lax.shift_right_logical(dsrt, 4), lax.bitwise_and(dsrt, 15)],
                    cnt,
                    mask=last,
                )

        pltpu.sync_copy(degv, out_hbm.at[c].at[s])

    return deg_kernel(dst, zdeg)


def _sc_aggregate(table, src, dst, zrows):
    mesh = plsc.VectorSubcoreMesh(core_axis_name="c", subcore_axis_name="s")

    @functools.partial(
        pl.kernel,
        out_type=jax.ShapeDtypeStruct((NC, N, D), jnp.float32),
        mesh=mesh,
        scratch_types=(
            [pltpu.VMEM((CH,), jnp.int32)] * 4
            + [pltpu.VMEM((CH,), jnp.int32)] * 4
            + [pltpu.VMEM((CH, D), jnp.float32)] * 2
            + [pltpu.SemaphoreType.DMA] * 12
            + [pltpu.VMEM_SHARED((ACC_ROWS, D), jnp.float32)]
        ),
    )
    def agg_kernel(table_hbm, src_hbm, dst_hbm, z_hbm, out_hbm,
                   sv0, sv1, sv2, sv3, dv0, dv1, dv2, dv3,
                   rows0, rows1,
                   is0, is1, is2, is3, js0, js1, js2, js3,
                   gsem0, gsem1, ssem0, ssem1, acc):
        c = lax.axis_index("c")
        s = lax.axis_index("s")
        wid = s * NC + c
        svs = (sv0, sv1, sv2, sv3)
        dvs = (dv0, dv1, dv2, dv3)
        isems = (is0, is1, is2, is3)
        jsems = (js0, js1, js2, js3)
        rows = (rows0, rows1)
        gsems = (gsem0, gsem1)
        ssems = (ssem0, ssem1)

        @pl.when(s < NS - 1)
        def _():
            pltpu.sync_copy(z_hbm.at[pl.ds(0, RPT)], acc.at[pl.ds(s * RPT, RPT)])

        @pl.when(s == NS - 1)
        def _():
            pltpu.sync_copy(z_hbm, acc.at[pl.ds((NS - 1) * RPT, RPT_LAST)])

        plsc.subcore_barrier()

        base = wid * EPW2

        def issue_idx(chunk, slot):
            off = base + chunk * CH
            pltpu.async_copy(src_hbm.at[pl.ds(off, CH)], svs[slot], isems[slot])
            pltpu.async_copy(dst_hbm.at[pl.ds(off, CH)], dvs[slot], jsems[slot])

        issue_idx(0, 0)
        issue_idx(1, 1)
        issue_idx(2, 2)
        pltpu.make_async_copy(src_hbm.at[pl.ds(0, CH)], sv0, is0).wait()
        pltpu.make_async_copy(dst_hbm.at[pl.ds(0, CH)], dv0, js0).wait()
        pltpu.async_copy(table_hbm.at[sv0], rows0, gsem0)

        @pl.loop(0, NCH, step=4)
        def _grp(k):
            for u in range(4):
                rb, rb1 = u % 2, (u + 1) % 2
                s4, s41, s43 = u, (u + 1) % 4, (u + 3) % 4

                pltpu.make_async_copy(table_hbm.at[svs[s4]],
                                      rows[rb], gsems[rb]).wait()
                pltpu.async_copy(rows[rb], acc.at[dvs[s4]], ssems[rb],
                                 add=True)

                def next_stage(s41=s41, s43=s43, rb1=rb1, u=u):
                    pltpu.make_async_copy(src_hbm.at[pl.ds(0, CH)],
                                          svs[s41], isems[s41]).wait()
                    pltpu.make_async_copy(dst_hbm.at[pl.ds(0, CH)],
                                          dvs[s41], jsems[s41]).wait()

                    def wait_prev(bb=rb1):
                        pltpu.make_async_copy(rows[bb], acc.at[dvs[bb]],
                                              ssems[bb]).wait()
                    if u == 0:
                        pl.when(k > 0)(wait_prev)
                    else:
                        wait_prev()

                    pltpu.async_copy(table_hbm.at[svs[s41]], rows[rb1],
                                     gsems[rb1])

                    def prefetch(ch=k + u + 3, slot=s43):
                        issue_idx(ch, slot)
                    if u == 0:
                        prefetch()
                    else:
                        pl.when(k < NCH - 4)(prefetch)

                if u < 3:
                    next_stage()
                else:
                    pl.when(k < NCH - 4)(next_stage)

        pltpu.make_async_copy(rows0, acc.at[dv2], ssem0).wait()
        pltpu.make_async_copy(rows1, acc.at[dv3], ssem1).wait()
        plsc.subcore_barrier()

        @pl.when(s < NS - 1)
        def _():
            pltpu.sync_copy(acc.at[pl.ds(s * RPT, RPT)],
                            out_hbm.at[c].at[pl.ds(s * RPT, RPT)])

        @pl.when(s == NS - 1)
        def _():
            pltpu.sync_copy(acc.at[pl.ds((NS - 1) * RPT, RPT_LAST)],
                            out_hbm.at[c].at[pl.ds((NS - 1) * RPT, RPT_LAST)])

    return agg_kernel(table, src, dst, zrows)


def _deg_factors(deg_ref):
    deg = jnp.sum(deg_ref[...], axis=1, keepdims=True) + 1.0
    dis = lax.rsqrt(deg)
    return dis, 1.0 / deg


def _tc_layer_in(x, degT, W, b2d):
    def body(x_ref, deg_ref, w_ref, b_ref, xws_ref, self_ref):
        dis, inv = _deg_factors(deg_ref)
        xw = jnp.dot(x_ref[...], w_ref[...],
                     preferred_element_type=jnp.float32, precision=_HI)
        xws_ref[...] = xw * dis
        self_ref[...] = xw * inv + b_ref[...]

    return pl.pallas_call(
        body,
        grid=(NBT,),
        in_specs=[
            pl.BlockSpec((BT, D), lambda i: (i, 0)),
            pl.BlockSpec((BT, NW), lambda i: (i, 0)),
            pl.BlockSpec((D, D), lambda i: (0, 0)),
            pl.BlockSpec((1, D), lambda i: (0, 0)),
        ],
        out_specs=[pl.BlockSpec((BT, D), lambda i: (i, 0)),
                   pl.BlockSpec((BT, D), lambda i: (i, 0))],
        out_shape=[jax.ShapeDtypeStruct((N, D), jnp.float32)] * 2,
    )(x, degT, W, b2d)


def _tc_layer_mid(p, degT, selfp, W, b2d):
    def body(p_ref, deg_ref, sp_ref, w_ref, b_ref, xws_ref, self_ref):
        dis, inv = _deg_factors(deg_ref)
        h = jnp.maximum((p_ref[0] + p_ref[1]) * dis + sp_ref[...], 0.0)
        xw = jnp.dot(h, w_ref[...],
                     preferred_element_type=jnp.float32, precision=_HI)
        xws_ref[...] = xw * dis
        self_ref[...] = xw * inv + b_ref[...]

    return pl.pallas_call(
        body,
        grid=(NBT,),
        in_specs=[
            pl.BlockSpec((NC, BT, D), lambda i: (0, i, 0)),
            pl.BlockSpec((BT, NW), lambda i: (i, 0)),
            pl.BlockSpec((BT, D), lambda i: (i, 0)),
            pl.BlockSpec((D, D), lambda i: (0, 0)),
            pl.BlockSpec((1, D), lambda i: (0, 0)),
        ],
        out_specs=[pl.BlockSpec((BT, D), lambda i: (i, 0)),
                   pl.BlockSpec((BT, D), lambda i: (i, 0))],
        out_shape=[jax.ShapeDtypeStruct((N, D), jnp.float32)] * 2,
    )(p, degT, selfp, W, b2d)


def _tc_final(q, degT, selfp, Wt, bt, Wh, bh, batch3):
    def body(q_ref, deg_ref, sp_ref, wt_ref, bt_ref, wh_ref, bh_ref, b_ref,
             h_ref, th_ref, hc_ref, lsm_ref, pooled):
        i = pl.program_id(0)
        dis, _ = _deg_factors(deg_ref)
        h2 = (q_ref[0] + q_ref[1]) * dis + sp_ref[...]
        h_ref[...] = h2
        theta = jnp.dot(h2, wt_ref[...],
                        preferred_element_type=jnp.float32, precision=_HI) + bt_ref[...]
        hc = jnp.dot(h2, wh_ref[...],
                     preferred_element_type=jnp.float32, precision=_HI) + bh_ref[...]
        th_ref[...] = theta[:, :C]
        hc_ref[...] = hc[:, :C]
        prod = theta * hc
        bb = b_ref[0]
        gi = lax.broadcasted_iota(jnp.int32, (G, BT), 0)
        oh = (gi == bb).astype(jnp.float32)
        part = jnp.dot(oh, prod, preferred_element_type=jnp.float32, precision=_HI)

        @pl.when(i == 0)
        def _():
            pooled[...] = jnp.zeros((G, D), jnp.float32)

        pooled[...] += part
        z = pooled[...]
        col = lax.broadcasted_iota(jnp.int32, (G, D), 1)
        zm = jnp.where(col < C, z, -jnp.inf)
        m = jnp.max(zm, axis=1, keepdims=True)
        ssum = jnp.sum(jnp.exp(zm - m), axis=1, keepdims=True)
        lsm_ref[...] = (z - m - jnp.log(ssum))[:, :C]

    return pl.pallas_call(
        body,
        grid=(NBT,),
        in_specs=[
            pl.BlockSpec((NC, BT, D), lambda i: (0, i, 0)),
            pl.BlockSpec((BT, NW), lambda i: (i, 0)),
            pl.BlockSpec((BT, D), lambda i: (i, 0)),
            pl.BlockSpec((D, D), lambda i: (0, 0)),
            pl.BlockSpec((1, D), lambda i: (0, 0)),
            pl.BlockSpec((D, D), lambda i: (0, 0)),
            pl.BlockSpec((1, D), lambda i: (0, 0)),
            pl.BlockSpec((1, 1, BT), lambda i: (i, 0, 0)),
        ],
        out_specs=[
            pl.BlockSpec((BT, D), lambda i: (i, 0)),
            pl.BlockSpec((BT, C), lambda i: (i, 0)),
            pl.BlockSpec((BT, C), lambda i: (i, 0)),
            pl.BlockSpec((G, C), lambda i: (0, 0)),
        ],
        out_shape=[
            jax.ShapeDtypeStruct((N, D), jnp.float32),
            jax.ShapeDtypeStruct((N, C), jnp.float32),
            jax.ShapeDtypeStruct((N, C), jnp.float32),
            jax.ShapeDtypeStruct((G, C), jnp.float32),
        ],
        scratch_shapes=[pltpu.VMEM((G, D), jnp.float32)],
    )(q, degT, selfp, Wt, bt, Wh, bh, batch3)


def kernel(x, edge_index, batch, W1, b1, W2, b2, W_theta, b_theta, W_h, b_h):
    f32 = jnp.float32
    src = edge_index[0].astype(jnp.int32)
    dst = edge_index[1].astype(jnp.int32)
    zdeg = jnp.zeros((DR, LANES), f32)
    zrows = jnp.zeros((RPT_LAST, D), f32)

    deg2 = _sc_degree(dst, zdeg)
    degT = deg2.reshape(NW, DR * LANES)[:, :N].T

    pad_src = jnp.broadcast_to(jnp.arange(EPAD, dtype=jnp.int32)[None, :],
                               (NW, EPAD))
    pad_dst = jnp.broadcast_to((N + jnp.arange(EPAD, dtype=jnp.int32))[None, :],
                               (NW, EPAD))
    src_p = jnp.concatenate([src.reshape(NW, EPW), pad_src], axis=1).reshape(-1)
    dst_p = jnp.concatenate([dst.reshape(NW, EPW), pad_dst], axis=1).reshape(-1)

    xws1, self1 = _tc_layer_in(x, degT, W1, b1.reshape(1, D))
    p = _sc_aggregate(xws1, src_p, dst_p, zrows)
    xws2, self2 = _tc_layer_mid(p, degT, self1, W2, b2.reshape(1, D))
    q = _sc_aggregate(xws2, src_p, dst_p, zrows)

    Wt = jnp.zeros((D, D), f32).at[:, :C].set(W_theta)
    Wh = jnp.zeros((D, D), f32).at[:, :C].set(W_h)
    bt = jnp.zeros((1, D), f32).at[0, :C].set(b_theta)
    bh = jnp.zeros((1, D), f32).at[0, :C].set(b_h)
    batch3 = batch.astype(jnp.int32).reshape(NBT, 1, BT)

    h, theta, hc, lsm = _tc_final(q, degT, self2, Wt, bt, Wh, bh, batch3)
    return lsm, h, theta, hc
